# Initial kernel scaffold; baseline (speedup 1.0000x reference)
import jax
import jax.numpy as jnp
from jax.experimental import pallas as pl


def kernel(x, edge_index, W_ge, b_ge, gene_emb, W_gin, bn_w, bn_b, W_k, b_k, W_q, W_v, b_v, W_e, b_e, W_d, b_d, W_p0, b_p0, W_p1, b_p1):
    raise NotImplementedError("write your pallas kernel here")


# speedup vs baseline: 3.8127x; 3.8127x over previous
import functools

import jax
import jax.numpy as jnp
from jax import lax
from jax.experimental import pallas as pl
from jax.experimental.pallas import tpu as pltpu
from jax.experimental.pallas import tpu_sc as plsc

_B = 2
_NG = 19836
_DIN = 11
_DH = 64
_N = _B * _NG
_E = 634752

_BN = 4408
_GRID = 9

_CH = 128
_EPW = 310 * _CH
_ROWS = 19968
_TRASH = _NG
_ZR = _ROWS // 16
_CPO = 1240
_CPO_LAST = _NG - 15 * _CPO


def _dot(a, b):
    return lax.dot_general(a, b, (((1,), (0,)), ((), ())),
                           preferred_element_type=jnp.float32)


def _gelu(t):
    return 0.5 * t * (1.0 + lax.erf(t * 0.7071067811865476))


def _enc_body(x_ref, wge_ref, bge_ref, emb_ref, o_ref):
    acc = lax.dot_general(x_ref[...], wge_ref[...], (((1,), (1,)), ((), ())),
                          preferred_element_type=jnp.float32)
    o_ref[...] = _gelu(acc + bge_ref[...]) + emb_ref[...]


_enc_call = pl.pallas_call(
    _enc_body,
    grid=(_GRID,),
    in_specs=[
        pl.BlockSpec((_BN, _DIN), lambda i: (i, 0)),
        pl.BlockSpec((_DH, _DIN), lambda i: (0, 0)),
        pl.BlockSpec((1, _DH), lambda i: (0, 0)),
        pl.BlockSpec((_BN, _DH), lambda i: (i, 0)),
    ],
    out_specs=pl.BlockSpec((_BN, _DH), lambda i: (i, 0)),
    out_shape=jax.ShapeDtypeStruct((_N, _DH), jnp.float32),
)


def _sc_segsum_body(h_hbm, src_hbm, dst_hbm, zeros_hbm, agg_hbm,
                    src_v, dst_v, idx_v, rows_v, acc_sh, sem):
    c = lax.axis_index("c")
    s = lax.axis_index("s")
    core_lo = c * _NG

    pltpu.sync_copy(zeros_hbm, acc_sh.at[pl.ds(s * _ZR, _ZR)])
    plsc.subcore_barrier()

    ebase = s * _EPW

    def chunk(ci):
        e0 = ebase + ci * _CH
        pltpu.sync_copy(src_hbm.at[pl.ds(e0, _CH)], src_v)
        pltpu.sync_copy(dst_hbm.at[pl.ds(e0, _CH)], dst_v)
        for j in range(_CH // 16):
            d = dst_v[pl.ds(j * 16, 16)]
            l = d - core_lo
            ok = (l >= 0) & (l < _NG)
            idx_v[0, pl.ds(j * 16, 16)] = jnp.where(ok, l, _TRASH)
        pltpu.async_copy(h_hbm.at[src_v], rows_v, sem).wait()
        pltpu.sync_copy(rows_v, acc_sh.at[idx_v.at[0]], add=True)

    def loop_body(ci, carry):
        chunk(ci)
        return carry

    lax.fori_loop(0, _EPW // _CH, loop_body, 0)

    plsc.subcore_barrier()

    @pl.when(s < 15)
    def _():
        lo = s * _CPO
        pltpu.sync_copy(acc_sh.at[pl.ds(lo, _CPO)],
                        agg_hbm.at[c, pl.ds(lo, _CPO)])

    @pl.when(s == 15)
    def _():
        lo = 15 * _CPO
        pltpu.sync_copy(acc_sh.at[pl.ds(lo, _CPO_LAST)],
                        agg_hbm.at[c, pl.ds(lo, _CPO_LAST)])


@functools.cache
def _get_sc_segsum():
    return functools.partial(
        pl.kernel,
        out_type=jax.ShapeDtypeStruct((_B, _NG, _DH), jnp.float32),
        mesh=plsc.VectorSubcoreMesh(core_axis_name="c", subcore_axis_name="s"),
        compiler_params=pltpu.CompilerParams(use_tc_tiling_on_sc=False),
        scratch_types=[
            pltpu.VMEM((_CH,), jnp.int32),
            pltpu.VMEM((_CH,), jnp.int32),
            pltpu.VMEM((1, _CH), jnp.int32),
            pltpu.VMEM((_CH, _DH), jnp.float32),
            pltpu.VMEM_SHARED((_ROWS, _DH), jnp.float32),
            pltpu.SemaphoreType.DMA,
        ],
    )(_sc_segsum_body)


def _stats_body(h_ref, a_ref, wgt_ref, bnw_ref, bnb_ref,
                scale_ref, shift_ref, sum_acc, sq_acc):
    i = pl.program_id(0)

    @pl.when(i == 0)
    def _():
        sum_acc[...] = jnp.zeros_like(sum_acc)
        sq_acc[...] = jnp.zeros_like(sq_acc)

    X = h_ref[...] + a_ref[...]
    sum_acc[...] += jnp.sum(X, axis=0, keepdims=True)
    sq_acc[...] += lax.dot_general(X, X, (((0,), (0,)), ((), ())),
                                   preferred_element_type=jnp.float32)

    @pl.when(i == _GRID - 1)
    def _():
        wgt = wgt_ref[...]
        m = _dot(sum_acc[...], wgt) / _N
        sw = _dot(sq_acc[...], wgt)
        ssq = jnp.sum(wgt * sw, axis=0, keepdims=True)
        var = ssq / _N - m * m
        scale = bnw_ref[...] * lax.rsqrt(var + 1e-5)
        scale_ref[...] = scale
        shift_ref[...] = bnb_ref[...] - m * scale


_stats_call = pl.pallas_call(
    _stats_body,
    grid=(_GRID,),
    in_specs=[
        pl.BlockSpec((_BN, _DH), lambda i: (i, 0)),
        pl.BlockSpec((_BN, _DH), lambda i: (i, 0)),
        pl.BlockSpec((_DH, _DH), lambda i: (0, 0)),
        pl.BlockSpec((1, _DH), lambda i: (0, 0)),
        pl.BlockSpec((1, _DH), lambda i: (0, 0)),
    ],
    out_specs=[
        pl.BlockSpec((1, _DH), lambda i: (0, 0)),
        pl.BlockSpec((1, _DH), lambda i: (0, 0)),
    ],
    out_shape=[
        jax.ShapeDtypeStruct((1, _DH), jnp.float32),
        jax.ShapeDtypeStruct((1, _DH), jnp.float32),
    ],
    scratch_shapes=[
        pltpu.VMEM((1, _DH), jnp.float32),
        pltpu.VMEM((_DH, _DH), jnp.float32),
    ],
)


def _main_body(h_ref, a_ref, scale_ref, shift_ref, wgt_ref,
               wkt_ref, bk_ref, wqt_ref, wvt_ref, bv_ref,
               wet_ref, be_ref, wdt_ref, bd_ref,
               wp0t_ref, bp0_ref, wp1t_ref, bp1_ref,
               w_out, dec_out, preds_out, loss_out,
               sse_acc, sabs_acc, gh_acc):
    i = pl.program_id(0)

    @pl.when(i == 0)
    def _():
        sse_acc[...] = jnp.zeros_like(sse_acc)
        sabs_acc[...] = jnp.zeros_like(sabs_acc)
        gh_acc[...] = jnp.zeros_like(gh_acc)

    X = h_ref[...] + a_ref[...]
    hg = _dot(X, wgt_ref[...])
    hb = _gelu(hg * scale_ref[...] + shift_ref[...])
    keys = _dot(hb, wkt_ref[...]) + bk_ref[...]
    w = jax.nn.sigmoid(_dot(keys, wqt_ref[...]))
    v = _dot(hb, wvt_ref[...]) + bv_ref[...]
    z = v * w
    enc = jnp.maximum(_dot(z, wet_ref[...]) + be_ref[...], 0.0)
    dec = _dot(enc, wdt_ref[...]) + bd_ref[...]
    w_out[...] = w
    dec_out[...] = dec
    sse_acc[...] += jnp.sum((dec - z) ** 2, keepdims=True)
    sabs_acc[...] += jnp.sum(jnp.abs(enc), keepdims=True)
    dw = dec * w
    row = lax.broadcasted_iota(jnp.int32, (_BN, 1), 0) + i * _BN
    m0 = (row < _NG).astype(jnp.float32)
    g0 = jnp.sum(dw * m0, axis=0, keepdims=True)
    g1 = jnp.sum(dw * (1.0 - m0), axis=0, keepdims=True)
    gh_acc[...] += jnp.concatenate([g0, g1], axis=0)

    @pl.when(i == _GRID - 1)
    def _():
        loss_out[...] = (sse_acc[...] + sabs_acc[...]) / (_N * _DH)
        p = _gelu(_dot(gh_acc[...], wp0t_ref[...]) + bp0_ref[...])
        preds_out[...] = _dot(p, wp1t_ref[...]) + bp1_ref[...]


def _rep(shape):
    return pl.BlockSpec(shape, lambda i: (0, 0))


_main_call = pl.pallas_call(
    _main_body,
    grid=(_GRID,),
    in_specs=[
        pl.BlockSpec((_BN, _DH), lambda i: (i, 0)),
        pl.BlockSpec((_BN, _DH), lambda i: (i, 0)),
        _rep((1, _DH)), _rep((1, _DH)), _rep((_DH, _DH)),
        _rep((_DH, _DH)), _rep((1, _DH)), _rep((_DH, 1)),
        _rep((_DH, _DH)), _rep((1, _DH)),
        _rep((_DH, _DH)), _rep((1, _DH)),
        _rep((_DH, _DH)), _rep((1, _DH)),
        _rep((_DH, _DH)), _rep((1, _DH)), _rep((_DH, 1)), _rep((1, 1)),
    ],
    out_specs=[
        pl.BlockSpec((_BN, 1), lambda i: (i, 0)),
        pl.BlockSpec((_BN, _DH), lambda i: (i, 0)),
        _rep((_B, 1)),
        _rep((1, 1)),
    ],
    out_shape=[
        jax.ShapeDtypeStruct((_N, 1), jnp.float32),
        jax.ShapeDtypeStruct((_N, _DH), jnp.float32),
        jax.ShapeDtypeStruct((_B, 1), jnp.float32),
        jax.ShapeDtypeStruct((1, 1), jnp.float32),
    ],
    scratch_shapes=[
        pltpu.VMEM((1, 1), jnp.float32),
        pltpu.VMEM((1, 1), jnp.float32),
        pltpu.VMEM((_B, _DH), jnp.float32),
    ],
)


def kernel(x, edge_index, W_ge, b_ge, gene_emb, W_gin, bn_w, bn_b,
           W_k, b_k, W_q, W_v, b_v, W_e, b_e, W_d, b_d,
           W_p0, b_p0, W_p1, b_p1):
    x2 = x.reshape(_N, _DIN)
    emb2 = jnp.concatenate([gene_emb, gene_emb], axis=0)
    h0 = _enc_call(x2, W_ge, b_ge.reshape(1, _DH), emb2)

    pad = 16 * _EPW - _E
    src = jnp.concatenate([edge_index[0], jnp.zeros((pad,), jnp.int32)])
    dst = jnp.concatenate([edge_index[1], jnp.full((pad,), _N, jnp.int32)])
    zeros = jnp.zeros((_ZR, _DH), jnp.float32)
    agg = _get_sc_segsum()(h0, src, dst, zeros).reshape(_N, _DH)

    wgt = W_gin.T
    scale, shift = _stats_call(h0, agg, wgt, bn_w.reshape(1, _DH),
                               bn_b.reshape(1, _DH))
    w, dec, preds, loss = _main_call(
        h0, agg, scale, shift, wgt,
        W_k.T, b_k.reshape(1, _DH), W_q.T,
        W_v.T, b_v.reshape(1, _DH),
        W_e.T, b_e.reshape(1, _DH),
        W_d.T, b_d.reshape(1, _DH),
        W_p0.T, b_p0.reshape(1, _DH), W_p1.T, b_p1.reshape(1, 1))
    return (preds, w, loss[0, 0], dec)


# speedup vs baseline: 7.0949x; 1.8609x over previous
import functools

import jax
import jax.numpy as jnp
from jax import lax
from jax.experimental import pallas as pl
from jax.experimental.pallas import tpu as pltpu
from jax.experimental.pallas import tpu_sc as plsc

_B = 2
_NG = 19836
_DIN = 11
_DH = 64
_DHH = 32
_N = _B * _NG
_E = 634752

_BN = 4408
_GRID = 9

_CH = 128
_NCB = 10
_SB = _NCB * _CH
_NBLK = 31
_EPW = _NBLK * _SB
_E2 = 16 * _EPW
_ROWS = _N + 8
_TRASH = _N
_ZR = _ROWS // 16
_CPO = 2480
_CPO_LAST = _N - 15 * _CPO


def _dot(a, b):
    return lax.dot_general(a.astype(jnp.bfloat16), b.astype(jnp.bfloat16),
                           (((1,), (0,)), ((), ())),
                           preferred_element_type=jnp.float32)


def _dot_x(a, b):
    return lax.dot_general(a, b, (((1,), (0,)), ((), ())),
                           preferred_element_type=jnp.float32,
                           precision=lax.Precision.HIGHEST)


def _dtd(a, b):
    return lax.dot_general(a.astype(jnp.bfloat16), b.astype(jnp.bfloat16),
                           (((0,), (0,)), ((), ())),
                           preferred_element_type=jnp.float32)


def _gelu(t):
    return 0.5 * t * (1.0 + lax.erf(t * 0.7071067811865476))


def _enc_body(x_ref, wlo_ref, whi_ref, blo_ref, bhi_ref,
              elo_ref, ehi_ref, o_ref):
    x = x_ref[...]
    xb = x.astype(jnp.bfloat16)
    lo = lax.dot_general(xb, wlo_ref[...].astype(jnp.bfloat16),
                         (((1,), (1,)), ((), ())),
                         preferred_element_type=jnp.float32)
    hi = lax.dot_general(xb, whi_ref[...].astype(jnp.bfloat16),
                         (((1,), (1,)), ((), ())),
                         preferred_element_type=jnp.float32)
    o_ref[0, :, :] = _gelu(lo + blo_ref[...]) + elo_ref[...]
    o_ref[1, :, :] = _gelu(hi + bhi_ref[...]) + ehi_ref[...]


_enc_call = pl.pallas_call(
    _enc_body,
    grid=(_GRID,),
    in_specs=[
        pl.BlockSpec((_BN, _DIN), lambda i: (i, 0)),
        pl.BlockSpec((_DHH, _DIN), lambda i: (0, 0)),
        pl.BlockSpec((_DHH, _DIN), lambda i: (0, 0)),
        pl.BlockSpec((1, _DHH), lambda i: (0, 0)),
        pl.BlockSpec((1, _DHH), lambda i: (0, 0)),
        pl.BlockSpec((_BN, _DHH), lambda i: (i, 0)),
        pl.BlockSpec((_BN, _DHH), lambda i: (i, 0)),
    ],
    out_specs=pl.BlockSpec((2, _BN, _DHH), lambda i: (0, i, 0)),
    out_shape=jax.ShapeDtypeStruct((2, _N, _DHH), jnp.float32),
)


def _sc_segsum_body(h_hbm, src_hbm, dst_hbm, zeros_hbm, agg_hbm,
                    srcs_v, dsts_v, idx_v, rows_v, acc_sh, sem0, sem1):
    c = lax.axis_index("c")
    s = lax.axis_index("s")

    pltpu.sync_copy(zeros_hbm, acc_sh.at[pl.ds(s * _ZR, _ZR)])
    plsc.subcore_barrier()

    rbase = s * _NBLK * _NCB
    tab_off = c * _N
    sems = (sem0, sem1)

    def gather(j, slot):
        return pltpu.make_async_copy(h_hbm.at[idx_v.at[j]],
                                     rows_v.at[slot], sems[slot])

    def block(blk, carry):
        r0 = rbase + blk * _NCB
        pltpu.sync_copy(src_hbm.at[pl.ds(r0, _NCB)], srcs_v)
        pltpu.sync_copy(dst_hbm.at[pl.ds(r0, _NCB)], dsts_v)
        for j in range(_NCB):
            for k in range(_CH // 16):
                idx_v[j, pl.ds(k * 16, 16)] = (
                    srcs_v[j, pl.ds(k * 16, 16)] + tab_off)
        gather(0, 0).start()
        gather(1, 1).start()
        for j in range(_NCB):
            slot = j % 2
            gather(j, slot).wait()
            pltpu.sync_copy(rows_v.at[slot], acc_sh.at[dsts_v.at[j]],
                            add=True)
            if j + 2 < _NCB:
                gather(j + 2, slot).start()
        return carry

    lax.fori_loop(0, _NBLK, block, 0)

    plsc.subcore_barrier()

    @pl.when(s < 15)
    def _():
        lo = s * _CPO
        pltpu.sync_copy(acc_sh.at[pl.ds(lo, _CPO)],
                        agg_hbm.at[c, pl.ds(lo, _CPO)])

    @pl.when(s == 15)
    def _():
        lo = 15 * _CPO
        pltpu.sync_copy(acc_sh.at[pl.ds(lo, _CPO_LAST)],
                        agg_hbm.at[c, pl.ds(lo, _CPO_LAST)])


@functools.cache
def _get_sc_segsum():
    return functools.partial(
        pl.kernel,
        out_type=jax.ShapeDtypeStruct((2, _N, _DHH), jnp.float32),
        mesh=plsc.VectorSubcoreMesh(core_axis_name="c", subcore_axis_name="s"),
        compiler_params=pltpu.CompilerParams(use_tc_tiling_on_sc=False),
        scratch_types=[
            pltpu.VMEM((_NCB, _CH), jnp.int32),
            pltpu.VMEM((_NCB, _CH), jnp.int32),
            pltpu.VMEM((_NCB, _CH), jnp.int32),
            pltpu.VMEM((2, _CH, _DHH), jnp.float32),
            pltpu.VMEM_SHARED((_ROWS, _DHH), jnp.float32),
            pltpu.SemaphoreType.DMA,
            pltpu.SemaphoreType.DMA,
        ],
    )(_sc_segsum_body)


def _stats_body(hlo_ref, hhi_ref, alo_ref, ahi_ref, wgtlo_ref, wgthi_ref,
                bnw_ref, bnb_ref, scale_ref, shift_ref,
                slo_acc, shi_acc, sll_acc, slh_acc, shl_acc, shh_acc):
    i = pl.program_id(0)

    @pl.when(i == 0)
    def _():
        slo_acc[...] = jnp.zeros_like(slo_acc)
        shi_acc[...] = jnp.zeros_like(shi_acc)
        sll_acc[...] = jnp.zeros_like(sll_acc)
        slh_acc[...] = jnp.zeros_like(slh_acc)
        shl_acc[...] = jnp.zeros_like(shl_acc)
        shh_acc[...] = jnp.zeros_like(shh_acc)

    Xlo = hlo_ref[0] + alo_ref[0]
    Xhi = hhi_ref[0] + ahi_ref[0]
    slo_acc[...] += jnp.sum(Xlo.astype(jnp.bfloat16).astype(jnp.float32),
                            axis=0, keepdims=True)
    shi_acc[...] += jnp.sum(Xhi.astype(jnp.bfloat16).astype(jnp.float32),
                            axis=0, keepdims=True)
    sll_acc[...] += _dtd(Xlo, Xlo)
    slh_acc[...] += _dtd(Xlo, Xhi)
    shl_acc[...] += _dtd(Xhi, Xlo)
    shh_acc[...] += _dtd(Xhi, Xhi)

    @pl.when(i == _GRID - 1)
    def _():
        wlo = wgtlo_ref[...].astype(jnp.bfloat16).astype(jnp.float32)
        whi = wgthi_ref[...].astype(jnp.bfloat16).astype(jnp.float32)
        m = (_dot_x(slo_acc[...], wlo) + _dot_x(shi_acc[...], whi)) / _N
        sw_top = _dot_x(sll_acc[...], wlo) + _dot_x(slh_acc[...], whi)
        sw_bot = _dot_x(shl_acc[...], wlo) + _dot_x(shh_acc[...], whi)
        ssq = (jnp.sum(wlo * sw_top, axis=0, keepdims=True)
               + jnp.sum(whi * sw_bot, axis=0, keepdims=True))
        var = ssq / _N - m * m
        scale = bnw_ref[...] * lax.rsqrt(var + 1e-5)
        scale_ref[...] = scale
        shift_ref[...] = bnb_ref[...] - m * scale


def _rep(shape):
    return pl.BlockSpec(shape, lambda i: (0, 0))


def _lo3d():
    return pl.BlockSpec((1, _BN, _DHH), lambda i: (0, i, 0))


def _hi3d():
    return pl.BlockSpec((1, _BN, _DHH), lambda i: (1, i, 0))


_stats_call = pl.pallas_call(
    _stats_body,
    grid=(_GRID,),
    in_specs=[
        _lo3d(), _hi3d(), _lo3d(), _hi3d(),
        _rep((_DHH, _DH)), _rep((_DHH, _DH)),
        _rep((1, _DH)), _rep((1, _DH)),
    ],
    out_specs=[
        _rep((1, _DH)),
        _rep((1, _DH)),
    ],
    out_shape=[
        jax.ShapeDtypeStruct((1, _DH), jnp.float32),
        jax.ShapeDtypeStruct((1, _DH), jnp.float32),
    ],
    scratch_shapes=[
        pltpu.VMEM((1, _DHH), jnp.float32),
        pltpu.VMEM((1, _DHH), jnp.float32),
        pltpu.VMEM((_DHH, _DHH), jnp.float32),
        pltpu.VMEM((_DHH, _DHH), jnp.float32),
        pltpu.VMEM((_DHH, _DHH), jnp.float32),
        pltpu.VMEM((_DHH, _DHH), jnp.float32),
    ],
)


def _main_body(hlo_ref, hhi_ref, alo_ref, ahi_ref, scale_ref, shift_ref,
               wgtlo_ref, wgthi_ref,
               wkt_ref, bk_ref, wqt_ref, wvt_ref, bv_ref,
               wet_ref, be_ref, wdt_ref, bd_ref,
               wp0t_ref, bp0_ref, wp1t_ref, bp1_ref,
               w_out, dec_out, preds_out, loss_out,
               sse_acc, sabs_acc, gh_acc):
    i = pl.program_id(0)

    @pl.when(i == 0)
    def _():
        sse_acc[...] = jnp.zeros_like(sse_acc)
        sabs_acc[...] = jnp.zeros_like(sabs_acc)
        gh_acc[...] = jnp.zeros_like(gh_acc)

    Xlo = hlo_ref[0] + alo_ref[0]
    Xhi = hhi_ref[0] + ahi_ref[0]
    hg = _dot(Xlo, wgtlo_ref[...]) + _dot(Xhi, wgthi_ref[...])
    hb = _gelu(hg * scale_ref[...] + shift_ref[...])
    keys = _dot(hb, wkt_ref[...]) + bk_ref[...]
    w = jax.nn.sigmoid(_dot(keys, wqt_ref[...]))
    v = _dot(hb, wvt_ref[...]) + bv_ref[...]
    z = v * w
    enc = jnp.maximum(_dot(z, wet_ref[...]) + be_ref[...], 0.0)
    dec = _dot(enc, wdt_ref[...]) + bd_ref[...]
    w_out[...] = w
    dec_out[...] = dec
    sse_acc[...] += jnp.sum((dec - z) ** 2, keepdims=True)
    sabs_acc[...] += jnp.sum(jnp.abs(enc), keepdims=True)
    dw = dec * w
    row = lax.broadcasted_iota(jnp.int32, (_BN, 1), 0) + i * _BN
    m0 = (row < _NG).astype(jnp.float32)
    g0 = jnp.sum(dw * m0, axis=0, keepdims=True)
    g1 = jnp.sum(dw * (1.0 - m0), axis=0, keepdims=True)
    gh_acc[...] += jnp.concatenate([g0, g1], axis=0)

    @pl.when(i == _GRID - 1)
    def _():
        loss_out[...] = (sse_acc[...] + sabs_acc[...]) / (_N * _DH)
        p = _gelu(_dot(gh_acc[...], wp0t_ref[...]) + bp0_ref[...])
        preds_out[...] = _dot(p, wp1t_ref[...]) + bp1_ref[...]


_main_call = pl.pallas_call(
    _main_body,
    grid=(_GRID,),
    in_specs=[
        _lo3d(), _hi3d(), _lo3d(), _hi3d(),
        _rep((1, _DH)), _rep((1, _DH)),
        _rep((_DHH, _DH)), _rep((_DHH, _DH)),
        _rep((_DH, _DH)), _rep((1, _DH)), _rep((_DH, 1)),
        _rep((_DH, _DH)), _rep((1, _DH)),
        _rep((_DH, _DH)), _rep((1, _DH)),
        _rep((_DH, _DH)), _rep((1, _DH)),
        _rep((_DH, _DH)), _rep((1, _DH)), _rep((_DH, 1)), _rep((1, 1)),
    ],
    out_specs=[
        pl.BlockSpec((_BN, 1), lambda i: (i, 0)),
        pl.BlockSpec((_BN, _DH), lambda i: (i, 0)),
        _rep((_B, 1)),
        _rep((1, 1)),
    ],
    out_shape=[
        jax.ShapeDtypeStruct((_N, 1), jnp.float32),
        jax.ShapeDtypeStruct((_N, _DH), jnp.float32),
        jax.ShapeDtypeStruct((_B, 1), jnp.float32),
        jax.ShapeDtypeStruct((1, 1), jnp.float32),
    ],
    scratch_shapes=[
        pltpu.VMEM((1, 1), jnp.float32),
        pltpu.VMEM((1, 1), jnp.float32),
        pltpu.VMEM((_B, _DH), jnp.float32),
    ],
)


def kernel(x, edge_index, W_ge, b_ge, gene_emb, W_gin, bn_w, bn_b,
           W_k, b_k, W_q, W_v, b_v, W_e, b_e, W_d, b_d,
           W_p0, b_p0, W_p1, b_p1):
    x2 = x.reshape(_N, _DIN)
    emb_lo = jnp.concatenate([gene_emb[:, :_DHH]] * _B, axis=0)
    emb_hi = jnp.concatenate([gene_emb[:, _DHH:]] * _B, axis=0)
    h2 = _enc_call(x2, W_ge[:_DHH], W_ge[_DHH:],
                   b_ge[:_DHH].reshape(1, _DHH), b_ge[_DHH:].reshape(1, _DHH),
                   emb_lo, emb_hi)

    pad = _E2 - _E
    src2 = jnp.concatenate(
        [edge_index[0], jnp.zeros((pad,), jnp.int32)]).reshape(-1, _CH)
    dst2 = jnp.concatenate(
        [edge_index[1], jnp.full((pad,), _N, jnp.int32)]).reshape(-1, _CH)
    zeros = jnp.zeros((_ZR, _DHH), jnp.float32)
    agg2 = _get_sc_segsum()(h2.reshape(2 * _N, _DHH), src2, dst2, zeros)

    wgt = W_gin.T
    wgt_lo, wgt_hi = wgt[:_DHH], wgt[_DHH:]
    scale, shift = _stats_call(h2, h2, agg2, agg2, wgt_lo, wgt_hi,
                               bn_w.reshape(1, _DH), bn_b.reshape(1, _DH))
    w, dec, preds, loss = _main_call(
        h2, h2, agg2, agg2, scale, shift, wgt_lo, wgt_hi,
        W_k.T, b_k.reshape(1, _DH), W_q.T,
        W_v.T, b_v.reshape(1, _DH),
        W_e.T, b_e.reshape(1, _DH),
        W_d.T, b_d.reshape(1, _DH),
        W_p0.T, b_p0.reshape(1, _DH), W_p1.T, b_p1.reshape(1, 1))
    return (preds, w, loss[0, 0], dec)
